# Initial kernel scaffold; baseline (speedup 1.0000x reference)
import jax
import jax.numpy as jnp
from jax.experimental import pallas as pl


def kernel(s0, edge_index, T):
    raise NotImplementedError("write your pallas kernel here")


# speedup vs baseline: 40.6951x; 40.6951x over previous
import functools

import jax
import jax.numpy as jnp
from jax import lax
from jax.experimental import pallas as pl
from jax.experimental.pallas import tpu as pltpu
from jax.experimental.pallas import tpu_sc as plsc

N = 100000
E = 6400000
STATE_N = 4
ROW_W = 8
ITERATIONS = 5

NUM_CORES = 2
NUM_SUBCORES = 16
NUM_W = NUM_CORES * NUM_SUBCORES

NODES_PER_W = 3136
NP = NODES_PER_W * NUM_W
ROWS_PER_TILE = NP // NUM_SUBCORES
STAGE_ROWS = ROWS_PER_TILE // 4
N_PAD_ROWS = NP - N
WORDS_PER_W = NODES_PER_W * STATE_N

CHUNK_ROWS = 8
CHUNK_E = CHUNK_ROWS * 128
N_CHUNKS_TOT = 6272
EP = N_CHUNKS_TOT * CHUNK_E
N_CHUNKS = N_CHUNKS_TOT // NUM_W

GROUPS_PER_W = NODES_PER_W // 4

_MESH2 = plsc.VectorSubcoreMesh(core_axis_name="c", subcore_axis_name="s")
_SC_PARAMS = pltpu.CompilerParams(use_tc_tiling_on_sc=False,
                                  needs_layout_passes=False)


def _softmax_body(t_ref, o_ref):
    x = t_ref[...]
    m = jnp.max(x, axis=-1, keepdims=True)
    e = jnp.exp(x - m)
    o_ref[...] = e / jnp.sum(e, axis=-1, keepdims=True)


def _softmax_tc(T):
    return pl.pallas_call(
        _softmax_body,
        out_shape=jax.ShapeDtypeStruct(T.shape, jnp.float32),
    )(T)


def _scatter_body(s_hbm, src_hbm, dst_hbm, zero_hbm, part_hbm,
                  acc_sh, stage_v, srcidx_v, dstidx_v, rows_v,
                  gsem, ssem):
    cid = lax.axis_index("c")
    sid = lax.axis_index("s")

    base = sid * ROWS_PER_TILE
    pltpu.sync_copy(zero_hbm, stage_v)
    for q in range(4):
        pltpu.sync_copy(stage_v, acc_sh.at[pl.ds(base + q * STAGE_ROWS,
                                                 STAGE_ROWS)])
    plsc.subcore_barrier()

    w = sid * NUM_CORES + cid
    chunk0 = w * N_CHUNKS

    @pl.loop(0, N_CHUNKS)
    def _chunk(ci):
        r = chunk0 + ci
        pltpu.sync_copy(src_hbm.at[r], srcidx_v)
        pltpu.sync_copy(dst_hbm.at[r], dstidx_v)
        gets = [
            pltpu.async_copy(s_hbm.at[srcidx_v.at[j]],
                             rows_v.at[pl.ds(j * 128, 128)], gsem)
            for j in range(CHUNK_ROWS)
        ]
        for cp in gets:
            cp.wait()
        puts = [
            pltpu.async_copy(rows_v.at[pl.ds(j * 128, 128)],
                             acc_sh.at[dstidx_v.at[j]], ssem, add=True)
            for j in range(CHUNK_ROWS)
        ]
        for cp in puts:
            cp.wait()

    plsc.subcore_barrier()

    for q in range(4):
        pltpu.sync_copy(acc_sh.at[pl.ds(base + q * STAGE_ROWS, STAGE_ROWS)],
                        stage_v)
        pltpu.sync_copy(stage_v,
                        part_hbm.at[cid, pl.ds(base + q * STAGE_ROWS,
                                               STAGE_ROWS)])


@functools.partial(
    pl.kernel,
    out_type=jax.ShapeDtypeStruct((NUM_CORES, NP, ROW_W), jnp.float32),
    mesh=_MESH2,
    scratch_types=[
        pltpu.VMEM_SHARED((NUM_CORES * NP, ROW_W), jnp.float32),
        pltpu.VMEM((STAGE_ROWS, ROW_W), jnp.float32),
        pltpu.VMEM((CHUNK_ROWS, 128), jnp.int32),
        pltpu.VMEM((CHUNK_ROWS, 128), jnp.int32),
        pltpu.VMEM((CHUNK_E, ROW_W), jnp.float32),
        pltpu.SemaphoreType.DMA,
        pltpu.SemaphoreType.DMA,
    ],
    compiler_params=_SC_PARAMS,
)
def _scatter_sc(s_hbm, src_hbm, dst_hbm, zero_hbm, part_hbm, *rest):
    _scatter_body(s_hbm, src_hbm, dst_hbm, zero_hbm, part_hbm, *rest)


def _lane_take(x, idx):
    dnums = lax.GatherDimensionNumbers(
        offset_dims=(), collapsed_slice_dims=(0,), start_index_map=(0,))
    return lax.gather(x, idx[:, None], dnums, slice_sizes=(1,),
                      mode=lax.GatherScatterMode.PROMISE_IN_BOUNDS)


def _transition_body(s_hbm, part_hbm, tm_hbm, out_hbm,
                     tm_v, a_v, b_v, s_v, o_v):
    cid = lax.axis_index("c")
    sid = lax.axis_index("s")
    w = sid * NUM_CORES + cid
    base = w * WORDS_PER_W

    pltpu.sync_copy(tm_hbm, tm_v)
    pltpu.sync_copy(part_hbm.at[0, pl.ds(base, WORDS_PER_W)], a_v)
    pltpu.sync_copy(part_hbm.at[1, pl.ds(base, WORDS_PER_W)], b_v)
    pltpu.sync_copy(s_hbm.at[pl.ds(base, WORDS_PER_W)], s_v)

    iota = lax.iota(jnp.int32, 16)
    pat_r = lax.shift_right_logical(iota, 2)
    pat_c = lax.bitwise_and(iota, 3)
    perm1 = lax.bitwise_xor(iota, 1)
    perm2 = lax.bitwise_xor(iota, 2)
    powv = lax.shift_left(jnp.int32(1), pat_c * 2).astype(jnp.float32)

    def _group(g, c):
        o = g * 16
        sums = a_v[pl.ds(o, 16)] + b_v[pl.ds(o, 16)]
        aggr = (jnp.where(sums > 0.5, 1.0, 0.0)
                + jnp.where(sums > 1.5, 1.0, 0.0)
                + jnp.where(sums > 2.5, 1.0, 0.0))
        wsum = aggr * powv
        t1 = wsum + _lane_take(wsum, perm1)
        num = t1 + _lane_take(t1, perm2)
        numi = num.astype(jnp.int32) * 16
        sv = s_v[pl.ds(o, 16)]
        acc = jnp.zeros((16,), jnp.float32)
        for st in range(STATE_N):
            coef = _lane_take(sv, pat_r * 4 + st)
            tmrow = plsc.load_gather(tm_v, [numi + (st * 4 + pat_c)])
            acc = acc + coef * tmrow
        o_v[pl.ds(o, 16)] = acc
        return c

    lax.fori_loop(0, GROUPS_PER_W, _group, 0)
    pltpu.sync_copy(o_v, out_hbm.at[pl.ds(base, WORDS_PER_W)])


@functools.partial(
    pl.kernel,
    out_type=jax.ShapeDtypeStruct((NP * STATE_N,), jnp.float32),
    mesh=_MESH2,
    scratch_types=[
        pltpu.VMEM((256 * 16,), jnp.float32),
        pltpu.VMEM((WORDS_PER_W,), jnp.float32),
        pltpu.VMEM((WORDS_PER_W,), jnp.float32),
        pltpu.VMEM((WORDS_PER_W,), jnp.float32),
        pltpu.VMEM((WORDS_PER_W,), jnp.float32),
    ],
    compiler_params=_SC_PARAMS,
)
def _transition_sc(s_hbm, part_hbm, tm_hbm, out_hbm, *rest):
    _transition_body(s_hbm, part_hbm, tm_hbm, out_hbm, *rest)


def kernel(s0, edge_index, T):
    Tm = _softmax_tc(T).reshape(256 * 16)

    src = edge_index[0]
    dst = edge_index[1]
    pad_idx = (jnp.arange(EP - E, dtype=jnp.int32) % N_PAD_ROWS) + N
    srcp = jnp.concatenate([src, pad_idx]).reshape(
        N_CHUNKS_TOT, CHUNK_ROWS, 128)
    dstp = jnp.concatenate([dst, pad_idx]).reshape(
        N_CHUNKS_TOT, CHUNK_ROWS, 128)
    s4 = jnp.pad(s0, ((0, NP - N), (0, 0)))
    zero = jnp.zeros((STAGE_ROWS, ROW_W), jnp.float32)

    for _ in range(ITERATIONS):
        s8 = jnp.pad(s4, ((0, 0), (0, ROW_W - STATE_N)))
        part8 = _scatter_sc(s8, srcp, dstp, zero)
        part4 = part8[:, :, :STATE_N].reshape(NUM_CORES, NP * STATE_N)
        s4 = _transition_sc(s4.reshape(NP * STATE_N), part4, Tm)
        s4 = s4.reshape(NP, STATE_N)
    return s4[:N]


# speedup vs baseline: 42.5412x; 1.0454x over previous
import functools

import jax
import jax.numpy as jnp
from jax import lax
from jax.experimental import pallas as pl
from jax.experimental.pallas import tpu as pltpu
from jax.experimental.pallas import tpu_sc as plsc

N = 100000
E = 6400000
STATE_N = 4
ROW_W = 8
ITERATIONS = 5

NUM_CORES = 2
NUM_SUBCORES = 16
NUM_W = NUM_CORES * NUM_SUBCORES

NODES_PER_W = 3136
NP = NODES_PER_W * NUM_W
ROWS_PER_TILE = NP // NUM_SUBCORES
STAGE_ROWS = ROWS_PER_TILE // 4
N_PAD_ROWS = NP - N
WORDS_PER_W = NODES_PER_W * STATE_N

CHUNK_ROWS = 8
CHUNK_E = CHUNK_ROWS * 128
N_CHUNKS_TOT = 6272
EP = N_CHUNKS_TOT * CHUNK_E
N_CHUNKS = N_CHUNKS_TOT // NUM_W

GROUPS_PER_W = NODES_PER_W // 4

_MESH2 = plsc.VectorSubcoreMesh(core_axis_name="c", subcore_axis_name="s")
_SC_PARAMS = pltpu.CompilerParams(use_tc_tiling_on_sc=False,
                                  needs_layout_passes=False)


def _softmax_body(t_ref, o_ref):
    x = t_ref[...]
    m = jnp.max(x, axis=-1, keepdims=True)
    e = jnp.exp(x - m)
    o_ref[...] = e / jnp.sum(e, axis=-1, keepdims=True)


def _softmax_tc(T):
    return pl.pallas_call(
        _softmax_body,
        out_shape=jax.ShapeDtypeStruct(T.shape, jnp.float32),
    )(T)


def _scatter_body(s_hbm, src_hbm, dst_hbm, zero_hbm, part_hbm,
                  acc_sh, stage_v, srcidx_v, dstidx_v, rows_v,
                  gsem, ssem):
    cid = lax.axis_index("c")
    sid = lax.axis_index("s")

    base = sid * ROWS_PER_TILE
    pltpu.sync_copy(zero_hbm, stage_v)
    for q in range(4):
        pltpu.sync_copy(stage_v, acc_sh.at[pl.ds(base + q * STAGE_ROWS,
                                                 STAGE_ROWS)])
    plsc.subcore_barrier()

    w = sid * NUM_CORES + cid
    chunk0 = w * N_CHUNKS

    @pl.loop(0, N_CHUNKS)
    def _chunk(ci):
        r = chunk0 + ci
        pltpu.sync_copy(src_hbm.at[r], srcidx_v)
        pltpu.sync_copy(dst_hbm.at[r], dstidx_v)
        gets = [
            pltpu.async_copy(s_hbm.at[srcidx_v.at[j]],
                             rows_v.at[pl.ds(j * 128, 128)], gsem)
            for j in range(CHUNK_ROWS)
        ]
        puts = []
        for j, cp in enumerate(gets):
            cp.wait()
            puts.append(
                pltpu.async_copy(rows_v.at[pl.ds(j * 128, 128)],
                                 acc_sh.at[dstidx_v.at[j]], ssem, add=True))
        for cp in puts:
            cp.wait()

    plsc.subcore_barrier()

    for q in range(4):
        pltpu.sync_copy(acc_sh.at[pl.ds(base + q * STAGE_ROWS, STAGE_ROWS)],
                        stage_v)
        pltpu.sync_copy(stage_v,
                        part_hbm.at[cid, pl.ds(base + q * STAGE_ROWS,
                                               STAGE_ROWS)])


@functools.partial(
    pl.kernel,
    out_type=jax.ShapeDtypeStruct((NUM_CORES, NP, ROW_W), jnp.float32),
    mesh=_MESH2,
    scratch_types=[
        pltpu.VMEM_SHARED((NUM_CORES * NP, ROW_W), jnp.float32),
        pltpu.VMEM((STAGE_ROWS, ROW_W), jnp.float32),
        pltpu.VMEM((CHUNK_ROWS, 128), jnp.int32),
        pltpu.VMEM((CHUNK_ROWS, 128), jnp.int32),
        pltpu.VMEM((CHUNK_E, ROW_W), jnp.float32),
        pltpu.SemaphoreType.DMA,
        pltpu.SemaphoreType.DMA,
    ],
    compiler_params=_SC_PARAMS,
)
def _scatter_sc(s_hbm, src_hbm, dst_hbm, zero_hbm, part_hbm, *rest):
    _scatter_body(s_hbm, src_hbm, dst_hbm, zero_hbm, part_hbm, *rest)


def _lane_take(x, idx):
    dnums = lax.GatherDimensionNumbers(
        offset_dims=(), collapsed_slice_dims=(0,), start_index_map=(0,))
    return lax.gather(x, idx[:, None], dnums, slice_sizes=(1,),
                      mode=lax.GatherScatterMode.PROMISE_IN_BOUNDS)


def _transition_body(s_hbm, part_hbm, tm_hbm, out_hbm,
                     tm_v, a_v, b_v, s_v, o_v):
    cid = lax.axis_index("c")
    sid = lax.axis_index("s")
    w = sid * NUM_CORES + cid
    base = w * WORDS_PER_W

    pltpu.sync_copy(tm_hbm, tm_v)
    pltpu.sync_copy(part_hbm.at[0, pl.ds(base, WORDS_PER_W)], a_v)
    pltpu.sync_copy(part_hbm.at[1, pl.ds(base, WORDS_PER_W)], b_v)
    pltpu.sync_copy(s_hbm.at[pl.ds(base, WORDS_PER_W)], s_v)

    iota = lax.iota(jnp.int32, 16)
    pat_r = lax.shift_right_logical(iota, 2)
    pat_c = lax.bitwise_and(iota, 3)
    perm1 = lax.bitwise_xor(iota, 1)
    perm2 = lax.bitwise_xor(iota, 2)
    powv = lax.shift_left(jnp.int32(1), pat_c * 2).astype(jnp.float32)

    def _group(g, c):
        o = g * 16
        sums = a_v[pl.ds(o, 16)] + b_v[pl.ds(o, 16)]
        aggr = (jnp.where(sums > 0.5, 1.0, 0.0)
                + jnp.where(sums > 1.5, 1.0, 0.0)
                + jnp.where(sums > 2.5, 1.0, 0.0))
        wsum = aggr * powv
        t1 = wsum + _lane_take(wsum, perm1)
        num = t1 + _lane_take(t1, perm2)
        numi = num.astype(jnp.int32) * 16
        sv = s_v[pl.ds(o, 16)]
        acc = jnp.zeros((16,), jnp.float32)
        for st in range(STATE_N):
            coef = _lane_take(sv, pat_r * 4 + st)
            tmrow = plsc.load_gather(tm_v, [numi + (st * 4 + pat_c)])
            acc = acc + coef * tmrow
        o_v[pl.ds(o, 16)] = acc
        return c

    lax.fori_loop(0, GROUPS_PER_W, _group, 0)
    pltpu.sync_copy(o_v, out_hbm.at[pl.ds(base, WORDS_PER_W)])


@functools.partial(
    pl.kernel,
    out_type=jax.ShapeDtypeStruct((NP * STATE_N,), jnp.float32),
    mesh=_MESH2,
    scratch_types=[
        pltpu.VMEM((256 * 16,), jnp.float32),
        pltpu.VMEM((WORDS_PER_W,), jnp.float32),
        pltpu.VMEM((WORDS_PER_W,), jnp.float32),
        pltpu.VMEM((WORDS_PER_W,), jnp.float32),
        pltpu.VMEM((WORDS_PER_W,), jnp.float32),
    ],
    compiler_params=_SC_PARAMS,
)
def _transition_sc(s_hbm, part_hbm, tm_hbm, out_hbm, *rest):
    _transition_body(s_hbm, part_hbm, tm_hbm, out_hbm, *rest)


def kernel(s0, edge_index, T):
    Tm = _softmax_tc(T).reshape(256 * 16)

    src = edge_index[0]
    dst = edge_index[1]
    pad_idx = (jnp.arange(EP - E, dtype=jnp.int32) % N_PAD_ROWS) + N
    srcp = jnp.concatenate([src, pad_idx]).reshape(
        N_CHUNKS_TOT, CHUNK_ROWS, 128)
    dstp = jnp.concatenate([dst, pad_idx]).reshape(
        N_CHUNKS_TOT, CHUNK_ROWS, 128)
    s4 = jnp.pad(s0, ((0, NP - N), (0, 0)))
    zero = jnp.zeros((STAGE_ROWS, ROW_W), jnp.float32)

    for _ in range(ITERATIONS):
        s8 = jnp.pad(s4, ((0, 0), (0, ROW_W - STATE_N)))
        part8 = _scatter_sc(s8, srcp, dstp, zero)
        part4 = part8[:, :, :STATE_N].reshape(NUM_CORES, NP * STATE_N)
        s4 = _transition_sc(s4.reshape(NP * STATE_N), part4, Tm)
        s4 = s4.reshape(NP, STATE_N)
    return s4[:N]


# speedup vs baseline: 54.3492x; 1.2776x over previous
import functools

import jax
import jax.numpy as jnp
from jax import lax
from jax.experimental import pallas as pl
from jax.experimental.pallas import tpu as pltpu
from jax.experimental.pallas import tpu_sc as plsc

N = 100000
E = 6400000
STATE_N = 4
ROW_W = 8
ITERATIONS = 5

NUM_CORES = 2
NUM_SUBCORES = 16
NUM_W = NUM_CORES * NUM_SUBCORES

NODES_PER_W = 3136
NP = NODES_PER_W * NUM_W
ROWS_PER_TILE = NP // NUM_SUBCORES
STAGE_ROWS = ROWS_PER_TILE // 4
N_PAD_ROWS = NP - N
WORDS_PER_W = NODES_PER_W * STATE_N

CHUNK_ROWS = 8
CHUNK_E = CHUNK_ROWS * 128
N_CHUNKS_TOT = 6272
EP = N_CHUNKS_TOT * CHUNK_E
N_CHUNKS = N_CHUNKS_TOT // NUM_W

GROUPS_PER_W = NODES_PER_W // 4

_MESH2 = plsc.VectorSubcoreMesh(core_axis_name="c", subcore_axis_name="s")
_SC_PARAMS = pltpu.CompilerParams(use_tc_tiling_on_sc=False,
                                  needs_layout_passes=False)


def _softmax_body(t_ref, o_ref):
    x = t_ref[...]
    m = jnp.max(x, axis=-1, keepdims=True)
    e = jnp.exp(x - m)
    o_ref[...] = e / jnp.sum(e, axis=-1, keepdims=True)


def _softmax_tc(T):
    return pl.pallas_call(
        _softmax_body,
        out_shape=jax.ShapeDtypeStruct(T.shape, jnp.float32),
    )(T)


def _scatter_body(s_hbm, src_hbm, dst_hbm, zero_hbm, part_hbm,
                  acc_sh, stage_v, srcidx_v, dstidx_v, rows_v,
                  gsem, ssem, isem):
    cid = lax.axis_index("c")
    sid = lax.axis_index("s")

    base = sid * ROWS_PER_TILE
    pltpu.sync_copy(zero_hbm, stage_v)
    for q in range(4):
        pltpu.sync_copy(stage_v, acc_sh.at[pl.ds(base + q * STAGE_ROWS,
                                                 STAGE_ROWS)])
    plsc.subcore_barrier()

    w = sid * NUM_CORES + cid
    chunk0 = w * N_CHUNKS

    pltpu.async_copy(src_hbm.at[chunk0], srcidx_v.at[0], isem)
    pltpu.async_copy(dst_hbm.at[chunk0], dstidx_v.at[0], isem)

    def _phase(slot, r_next):
        pltpu.make_async_copy(src_hbm.at[0], srcidx_v.at[slot], isem).wait()
        pltpu.make_async_copy(dst_hbm.at[0], dstidx_v.at[slot], isem).wait()
        pltpu.async_copy(src_hbm.at[r_next], srcidx_v.at[1 - slot], isem)
        pltpu.async_copy(dst_hbm.at[r_next], dstidx_v.at[1 - slot], isem)
        gets = [
            pltpu.async_copy(s_hbm.at[srcidx_v.at[slot, j]],
                             rows_v.at[pl.ds(j * 128, 128)], gsem)
            for j in range(CHUNK_ROWS)
        ]
        puts = []
        for j, cp in enumerate(gets):
            cp.wait()
            puts.append(
                pltpu.async_copy(rows_v.at[pl.ds(j * 128, 128)],
                                 acc_sh.at[dstidx_v.at[slot, j]], ssem,
                                 add=True))
        for cp in puts:
            cp.wait()

    last = jnp.int32(N_CHUNKS_TOT - 1)

    @pl.loop(0, N_CHUNKS // 2)
    def _pair(ck):
        r = chunk0 + ck * 2
        _phase(0, jnp.minimum(r + 1, last))
        _phase(1, jnp.minimum(r + 2, last))

    plsc.subcore_barrier()

    for q in range(4):
        pltpu.sync_copy(acc_sh.at[pl.ds(base + q * STAGE_ROWS, STAGE_ROWS)],
                        stage_v)
        pltpu.sync_copy(stage_v,
                        part_hbm.at[cid, pl.ds(base + q * STAGE_ROWS,
                                               STAGE_ROWS)])


@functools.partial(
    pl.kernel,
    out_type=jax.ShapeDtypeStruct((NUM_CORES, NP, ROW_W), jnp.float32),
    mesh=_MESH2,
    scratch_types=[
        pltpu.VMEM_SHARED((NUM_CORES * NP, ROW_W), jnp.float32),
        pltpu.VMEM((STAGE_ROWS, ROW_W), jnp.float32),
        pltpu.VMEM((2, CHUNK_ROWS, 128), jnp.int32),
        pltpu.VMEM((2, CHUNK_ROWS, 128), jnp.int32),
        pltpu.VMEM((CHUNK_E, ROW_W), jnp.float32),
        pltpu.SemaphoreType.DMA,
        pltpu.SemaphoreType.DMA,
        pltpu.SemaphoreType.DMA,
    ],
    compiler_params=_SC_PARAMS,
)
def _scatter_sc(s_hbm, src_hbm, dst_hbm, zero_hbm, part_hbm, *rest):
    _scatter_body(s_hbm, src_hbm, dst_hbm, zero_hbm, part_hbm, *rest)


def _lane_take(x, idx):
    dnums = lax.GatherDimensionNumbers(
        offset_dims=(), collapsed_slice_dims=(0,), start_index_map=(0,))
    return lax.gather(x, idx[:, None], dnums, slice_sizes=(1,),
                      mode=lax.GatherScatterMode.PROMISE_IN_BOUNDS)


def _transition_body(s_hbm, part_hbm, tm_hbm, out_hbm,
                     tm_v, a_v, b_v, s_v, o_v):
    cid = lax.axis_index("c")
    sid = lax.axis_index("s")
    w = sid * NUM_CORES + cid
    base = w * WORDS_PER_W

    pltpu.sync_copy(tm_hbm, tm_v)
    pltpu.sync_copy(part_hbm.at[0, pl.ds(base, WORDS_PER_W)], a_v)
    pltpu.sync_copy(part_hbm.at[1, pl.ds(base, WORDS_PER_W)], b_v)
    pltpu.sync_copy(s_hbm.at[pl.ds(base, WORDS_PER_W)], s_v)

    iota = lax.iota(jnp.int32, 16)
    pat_r = lax.shift_right_logical(iota, 2)
    pat_c = lax.bitwise_and(iota, 3)
    perm1 = lax.bitwise_xor(iota, 1)
    perm2 = lax.bitwise_xor(iota, 2)
    powv = lax.shift_left(jnp.int32(1), pat_c * 2).astype(jnp.float32)

    def _group(g, c):
        o = g * 16
        sums = a_v[pl.ds(o, 16)] + b_v[pl.ds(o, 16)]
        aggr = (jnp.where(sums > 0.5, 1.0, 0.0)
                + jnp.where(sums > 1.5, 1.0, 0.0)
                + jnp.where(sums > 2.5, 1.0, 0.0))
        wsum = aggr * powv
        t1 = wsum + _lane_take(wsum, perm1)
        num = t1 + _lane_take(t1, perm2)
        numi = num.astype(jnp.int32) * 16
        sv = s_v[pl.ds(o, 16)]
        acc = jnp.zeros((16,), jnp.float32)
        for st in range(STATE_N):
            coef = _lane_take(sv, pat_r * 4 + st)
            tmrow = plsc.load_gather(tm_v, [numi + (st * 4 + pat_c)])
            acc = acc + coef * tmrow
        o_v[pl.ds(o, 16)] = acc
        return c

    lax.fori_loop(0, GROUPS_PER_W, _group, 0)
    pltpu.sync_copy(o_v, out_hbm.at[pl.ds(base, WORDS_PER_W)])


@functools.partial(
    pl.kernel,
    out_type=jax.ShapeDtypeStruct((NP * STATE_N,), jnp.float32),
    mesh=_MESH2,
    scratch_types=[
        pltpu.VMEM((256 * 16,), jnp.float32),
        pltpu.VMEM((WORDS_PER_W,), jnp.float32),
        pltpu.VMEM((WORDS_PER_W,), jnp.float32),
        pltpu.VMEM((WORDS_PER_W,), jnp.float32),
        pltpu.VMEM((WORDS_PER_W,), jnp.float32),
    ],
    compiler_params=_SC_PARAMS,
)
def _transition_sc(s_hbm, part_hbm, tm_hbm, out_hbm, *rest):
    _transition_body(s_hbm, part_hbm, tm_hbm, out_hbm, *rest)


def kernel(s0, edge_index, T):
    Tm = _softmax_tc(T).reshape(256 * 16)

    src = edge_index[0]
    dst = edge_index[1]
    pad_idx = (jnp.arange(EP - E, dtype=jnp.int32) % N_PAD_ROWS) + N
    srcp = jnp.concatenate([src, pad_idx]).reshape(
        N_CHUNKS_TOT, CHUNK_ROWS, 128)
    dstp = jnp.concatenate([dst, pad_idx]).reshape(
        N_CHUNKS_TOT, CHUNK_ROWS, 128)
    s4 = jnp.pad(s0, ((0, NP - N), (0, 0)))
    zero = jnp.zeros((STAGE_ROWS, ROW_W), jnp.float32)

    for _ in range(ITERATIONS):
        s8 = jnp.pad(s4, ((0, 0), (0, ROW_W - STATE_N)))
        part8 = _scatter_sc(s8, srcp, dstp, zero)
        part4 = part8[:, :, :STATE_N].reshape(NUM_CORES, NP * STATE_N)
        s4 = _transition_sc(s4.reshape(NP * STATE_N), part4, Tm)
        s4 = s4.reshape(NP, STATE_N)
    return s4[:N]
